# Initial kernel scaffold; baseline (speedup 1.0000x reference)
import jax
import jax.numpy as jnp
from jax.experimental import pallas as pl


def kernel(feature, edge_index, weight_b, weight_a, G):
    raise NotImplementedError("write your pallas kernel here")


# speedup vs baseline: 7.4188x; 7.4188x over previous
import functools

import jax
import jax.numpy as jnp
from jax import lax
from jax.experimental import pallas as pl
from jax.experimental.pallas import tpu as pltpu
from jax.experimental.pallas import tpu_sc as plsc

N = 10000
D_FEAT = 128
DIM = 64
Q = 3
N_LAYERS = 4
E_PER = 320000

CHUNK = 128
CHUNKS_PER_LAYER = E_PER // CHUNK
NC, NS = 2, 16
CPT = 160
CHUNKS_PAD = CPT * NS
ACC_ROWS = 10240
ZROWS = ACC_ROWS // NS


def _proj_body(f_ref, g_ref, o_ref):
    f = f_ref[...]
    ss = jnp.sum(f * f, axis=1, keepdims=True)
    fn = f / (jnp.sqrt(ss) + 1e-12)
    o_ref[...] = jnp.dot(fn, g_ref[...], preferred_element_type=jnp.float32)


def _project(feature, G):
    blk = 1000
    grid = N // blk
    return pl.pallas_call(
        _proj_body,
        grid=(grid,),
        in_specs=[
            pl.BlockSpec((blk, D_FEAT), lambda i: (i, 0)),
            pl.BlockSpec((D_FEAT, DIM), lambda i: (0, 0)),
        ],
        out_specs=pl.BlockSpec((blk, DIM), lambda i: (i, 0)),
        out_shape=jax.ShapeDtypeStruct((N, DIM), jnp.float32),
    )(feature, G)


def _hop_body(u_hbm, src_hbm, dst_hbm, zeros_hbm, p_hbm,
              sidx, didx, rows, acc0, acc1, sem):
    c = lax.axis_index("c")
    t = lax.axis_index("s")

    pltpu.sync_copy(zeros_hbm, acc0.at[pl.ds(t * ZROWS, ZROWS)])
    pltpu.sync_copy(zeros_hbm, acc1.at[pl.ds(t * ZROWS, ZROWS)])
    plsc.subcore_barrier()

    for ll in range(2):
        acc = acc0 if ll == 0 else acc1
        layer = c * 2 + ll
        pltpu.sync_copy(src_hbm.at[layer, pl.ds(t * CPT, CPT)], sidx)
        pltpu.sync_copy(dst_hbm.at[layer, pl.ds(t * CPT, CPT)], didx)

        def chunk_body(j, carry, acc=acc):
            pltpu.async_copy(u_hbm.at[sidx.at[j]], rows, sem).wait()
            pltpu.sync_copy(rows, acc.at[didx.at[j]], add=True)
            return carry

        lax.fori_loop(0, CPT, chunk_body, 0)

    plsc.subcore_barrier()
    pltpu.sync_copy(acc0.at[pl.ds(t * ZROWS, ZROWS)],
                    p_hbm.at[c * 2, pl.ds(t * ZROWS, ZROWS)])
    pltpu.sync_copy(acc1.at[pl.ds(t * ZROWS, ZROWS)],
                    p_hbm.at[c * 2 + 1, pl.ds(t * ZROWS, ZROWS)])


_hop = functools.partial(
    pl.kernel,
    _hop_body,
    out_type=jax.ShapeDtypeStruct((N_LAYERS, ACC_ROWS, DIM), jnp.float32),
    mesh=plsc.VectorSubcoreMesh(core_axis_name="c", subcore_axis_name="s"),
    compiler_params=pltpu.CompilerParams(use_tc_tiling_on_sc=False),
    scratch_types=[
        pltpu.VMEM((CPT, CHUNK), jnp.int32),
        pltpu.VMEM((CPT, CHUNK), jnp.int32),
        pltpu.VMEM((CHUNK, DIM), jnp.float32),
        pltpu.VMEM_SHARED((ACC_ROWS, DIM), jnp.float32),
        pltpu.VMEM_SHARED((ACC_ROWS, DIM), jnp.float32),
        pltpu.SemaphoreType.DMA,
    ],
)()


def _merge_body(p_ref, wb_ref, o_ref):
    o_ref[...] = (wb_ref[0, 0] * p_ref[0] + wb_ref[1, 0] * p_ref[1]
                  + wb_ref[2, 0] * p_ref[2] + wb_ref[3, 0] * p_ref[3])


def _merge(P, wb):
    blk = 1000
    return pl.pallas_call(
        _merge_body,
        grid=(N // blk,),
        in_specs=[
            pl.BlockSpec((N_LAYERS, blk, DIM), lambda i: (0, i, 0)),
            pl.BlockSpec(memory_space=pltpu.SMEM),
        ],
        out_specs=pl.BlockSpec((blk, DIM), lambda i: (i, 0)),
        out_shape=jax.ShapeDtypeStruct((N, DIM), jnp.float32),
    )(P, wb)


def _merge_final_body(p_ref, wb_ref, wa_ref, u1_ref, u2_ref, o_ref):
    u3 = (wb_ref[0, 0] * p_ref[0] + wb_ref[1, 0] * p_ref[1]
          + wb_ref[2, 0] * p_ref[2] + wb_ref[3, 0] * p_ref[3])
    o_ref[...] = (wa_ref[0, 0] * u1_ref[...] + wa_ref[1, 0] * u2_ref[...]
                  + wa_ref[2, 0] * u3)


def _merge_final(P, wb, wa, U1, U2):
    blk = 1000
    return pl.pallas_call(
        _merge_final_body,
        grid=(N // blk,),
        in_specs=[
            pl.BlockSpec((N_LAYERS, blk, DIM), lambda i: (0, i, 0)),
            pl.BlockSpec(memory_space=pltpu.SMEM),
            pl.BlockSpec(memory_space=pltpu.SMEM),
            pl.BlockSpec((blk, DIM), lambda i: (i, 0)),
            pl.BlockSpec((blk, DIM), lambda i: (i, 0)),
        ],
        out_specs=pl.BlockSpec((blk, DIM), lambda i: (i, 0)),
        out_shape=jax.ShapeDtypeStruct((N, DIM), jnp.float32),
    )(P, wb, wa, U1, U2)


def kernel(feature, edge_index, weight_b, weight_a, G):
    src = edge_index[:, 0, :].reshape(N_LAYERS, CHUNKS_PER_LAYER, CHUNK)
    dst = edge_index[:, 1, :].reshape(N_LAYERS, CHUNKS_PER_LAYER, CHUNK)
    pad = CHUNKS_PAD - CHUNKS_PER_LAYER
    src = jnp.pad(src, ((0, 0), (0, pad), (0, 0)))
    dst = jnp.pad(dst, ((0, 0), (0, pad), (0, 0)), constant_values=N)
    zeros = jnp.zeros((ZROWS, DIM), jnp.float32)

    U = _project(feature, G)
    P1 = _hop(U, src, dst, zeros)
    U1 = _merge(P1, weight_b)
    P2 = _hop(U1, src, dst, zeros)
    U2 = _merge(P2, weight_b)
    P3 = _hop(U2, src, dst, zeros)
    return _merge_final(P3, weight_b, weight_a, U1, U2)


# speedup vs baseline: 8.7012x; 1.1729x over previous
import functools

import jax
import jax.numpy as jnp
from jax import lax
from jax.experimental import pallas as pl
from jax.experimental.pallas import tpu as pltpu
from jax.experimental.pallas import tpu_sc as plsc

N = 10000
D_FEAT = 128
DIM = 64
Q = 3
N_LAYERS = 4
E_PER = 320000

CHUNK = 128
CHUNKS_PER_LAYER = E_PER // CHUNK
NC, NS = 2, 16
CPT = 160
CHUNKS_PAD = CPT * NS
ACC_ROWS = 10240
ZROWS = ACC_ROWS // NS


def _proj_body(f_ref, g_ref, o_ref):
    f = f_ref[...]
    ss = jnp.sum(f * f, axis=1, keepdims=True)
    fn = f / (jnp.sqrt(ss) + 1e-12)
    o_ref[...] = jnp.dot(fn, g_ref[...], preferred_element_type=jnp.float32)


def _project(feature, G):
    blk = 1000
    grid = N // blk
    return pl.pallas_call(
        _proj_body,
        grid=(grid,),
        in_specs=[
            pl.BlockSpec((blk, D_FEAT), lambda i: (i, 0)),
            pl.BlockSpec((D_FEAT, DIM), lambda i: (0, 0)),
        ],
        out_specs=pl.BlockSpec((blk, DIM), lambda i: (i, 0)),
        out_shape=jax.ShapeDtypeStruct((N, DIM), jnp.float32),
    )(feature, G)


NBUF = 4
IG = 32


def _hop_body(u_hbm, src_hbm, dst_hbm, zeros_hbm, p_hbm,
              sidx, didx, rows, acc0, acc1, gsem):
    c = lax.axis_index("c")
    t = lax.axis_index("s")

    pltpu.sync_copy(zeros_hbm, acc0.at[pl.ds(t * ZROWS, ZROWS)])
    pltpu.sync_copy(zeros_hbm, acc1.at[pl.ds(t * ZROWS, ZROWS)])
    plsc.subcore_barrier()

    for ll in range(2):
        acc = acc0 if ll == 0 else acc1
        layer = c * 2 + ll
        for ig in range(CPT // IG):
            base = t * CPT + ig * IG
            pltpu.sync_copy(src_hbm.at[layer, pl.ds(base, IG)], sidx)
            pltpu.sync_copy(dst_hbm.at[layer, pl.ds(base, IG)], didx)

            for b in range(NBUF):
                pltpu.async_copy(u_hbm.at[sidx.at[b]], rows.at[b],
                                 gsem.at[b])

            def group_body(g, carry, acc=acc):
                for b in range(NBUF):
                    j = g * NBUF + b
                    pltpu.make_async_copy(
                        u_hbm.at[sidx.at[j]], rows.at[b], gsem.at[b]).wait()
                    pltpu.sync_copy(rows.at[b], acc.at[didx.at[j]],
                                    add=True)

                    @pl.when(j + NBUF < IG)
                    def _(j=j, b=b):
                        pltpu.async_copy(
                            u_hbm.at[sidx.at[j + NBUF]], rows.at[b],
                            gsem.at[b])
                return carry

            lax.fori_loop(0, IG // NBUF, group_body, 0)

    plsc.subcore_barrier()
    pltpu.sync_copy(acc0.at[pl.ds(t * ZROWS, ZROWS)],
                    p_hbm.at[c * 2, pl.ds(t * ZROWS, ZROWS)])
    pltpu.sync_copy(acc1.at[pl.ds(t * ZROWS, ZROWS)],
                    p_hbm.at[c * 2 + 1, pl.ds(t * ZROWS, ZROWS)])


_hop = functools.partial(
    pl.kernel,
    _hop_body,
    out_type=jax.ShapeDtypeStruct((N_LAYERS, ACC_ROWS, DIM), jnp.float32),
    mesh=plsc.VectorSubcoreMesh(core_axis_name="c", subcore_axis_name="s"),
    compiler_params=pltpu.CompilerParams(use_tc_tiling_on_sc=False),
    scratch_types=[
        pltpu.VMEM((IG, CHUNK), jnp.int32),
        pltpu.VMEM((IG, CHUNK), jnp.int32),
        pltpu.VMEM((NBUF, CHUNK, DIM), jnp.float32),
        pltpu.VMEM_SHARED((ACC_ROWS, DIM), jnp.float32),
        pltpu.VMEM_SHARED((ACC_ROWS, DIM), jnp.float32),
        pltpu.SemaphoreType.DMA((NBUF,)),
    ],
)()


def _merge_body(p_ref, wb_ref, o_ref):
    o_ref[...] = (wb_ref[0, 0] * p_ref[0] + wb_ref[1, 0] * p_ref[1]
                  + wb_ref[2, 0] * p_ref[2] + wb_ref[3, 0] * p_ref[3])


def _merge(P, wb):
    blk = 1000
    return pl.pallas_call(
        _merge_body,
        grid=(N // blk,),
        in_specs=[
            pl.BlockSpec((N_LAYERS, blk, DIM), lambda i: (0, i, 0)),
            pl.BlockSpec(memory_space=pltpu.SMEM),
        ],
        out_specs=pl.BlockSpec((blk, DIM), lambda i: (i, 0)),
        out_shape=jax.ShapeDtypeStruct((N, DIM), jnp.float32),
    )(P, wb)


def _merge_final_body(p_ref, wb_ref, wa_ref, u1_ref, u2_ref, o_ref):
    u3 = (wb_ref[0, 0] * p_ref[0] + wb_ref[1, 0] * p_ref[1]
          + wb_ref[2, 0] * p_ref[2] + wb_ref[3, 0] * p_ref[3])
    o_ref[...] = (wa_ref[0, 0] * u1_ref[...] + wa_ref[1, 0] * u2_ref[...]
                  + wa_ref[2, 0] * u3)


def _merge_final(P, wb, wa, U1, U2):
    blk = 1000
    return pl.pallas_call(
        _merge_final_body,
        grid=(N // blk,),
        in_specs=[
            pl.BlockSpec((N_LAYERS, blk, DIM), lambda i: (0, i, 0)),
            pl.BlockSpec(memory_space=pltpu.SMEM),
            pl.BlockSpec(memory_space=pltpu.SMEM),
            pl.BlockSpec((blk, DIM), lambda i: (i, 0)),
            pl.BlockSpec((blk, DIM), lambda i: (i, 0)),
        ],
        out_specs=pl.BlockSpec((blk, DIM), lambda i: (i, 0)),
        out_shape=jax.ShapeDtypeStruct((N, DIM), jnp.float32),
    )(P, wb, wa, U1, U2)


def kernel(feature, edge_index, weight_b, weight_a, G):
    src = edge_index[:, 0, :].reshape(N_LAYERS, CHUNKS_PER_LAYER, CHUNK)
    dst = edge_index[:, 1, :].reshape(N_LAYERS, CHUNKS_PER_LAYER, CHUNK)
    pad = CHUNKS_PAD - CHUNKS_PER_LAYER
    src = jnp.pad(src, ((0, 0), (0, pad), (0, 0)))
    dst = jnp.pad(dst, ((0, 0), (0, pad), (0, 0)), constant_values=N)
    zeros = jnp.zeros((ZROWS, DIM), jnp.float32)

    U = _project(feature, G)
    P1 = _hop(U, src, dst, zeros)
    U1 = _merge(P1, weight_b)
    P2 = _hop(U1, src, dst, zeros)
    U2 = _merge(P2, weight_b)
    P3 = _hop(U2, src, dst, zeros)
    return _merge_final(P3, weight_b, weight_a, U1, U2)
